# baseline (speedup 1.0000x reference)
import jax
import jax.numpy as jnp
from jax.experimental import pallas as pl
from jax.experimental.pallas import tpu as pltpu

_EPS = 1e-8
_B = 4096
_D = 256
_BLK = 512
_R = _B // _BLK
_CH = 512
_NCH = _B // _CH


def _norm_body(x_ref, y_ref):
    x = x_ref[...]
    nrm2 = jax.lax.dot_general(
        x * x, jnp.ones((_D, 1), jnp.float32), (((1,), (0,)), ((), ())),
        preferred_element_type=jnp.float32)
    inv = 1.0 / jnp.maximum(jnp.sqrt(nrm2), _EPS)
    y_ref[...] = (x * inv).astype(jnp.bfloat16)


def _koleo_body(y_ref, acc_ref):
    i = pl.program_id(0)

    xi = y_ref[pl.ds(i * _BLK, _BLK), :]
    diag = (jax.lax.broadcasted_iota(jnp.int32, (_BLK, _CH), 0)
            == jax.lax.broadcasted_iota(jnp.int32, (_BLK, _CH), 1))

    mlanes = jnp.full((_BLK, 128), jnp.float32(-3e38))
    for kk in range(_NCH):
        k = jax.lax.rem(i + kk, _NCH)
        xc = y_ref[pl.ds(k * _CH, _CH), :]
        dch = jax.lax.dot_general(
            xi, xc, (((1,), (1,)), ((), ())),
            preferred_element_type=jnp.float32)
        if kk == 0:
            dch = jnp.where(diag, jnp.float32(-1.0), dch)
        m4 = jnp.maximum(
            jnp.maximum(dch[:, 0:128], dch[:, 128:256]),
            jnp.maximum(dch[:, 256:384], dch[:, 384:512]))
        mlanes = jnp.maximum(mlanes, m4)
    m = jnp.max(mlanes, axis=1, keepdims=True)

    dist2 = 2.0 + _D * _EPS * _EPS - 2.0 * m
    dist = jnp.sqrt(jnp.maximum(dist2, 0.0))
    part = jnp.sum(jnp.log(dist + _EPS))

    @pl.when(i == 0)
    def _first():
        acc_ref[0, 0] = part

    @pl.when(i > 0)
    def _rest():
        acc_ref[0, 0] = acc_ref[0, 0] + part


def kernel(student_output):
    y = pl.pallas_call(
        _norm_body,
        out_shape=jax.ShapeDtypeStruct((_B, _D), jnp.bfloat16),
    )(student_output)
    acc = pl.pallas_call(
        _koleo_body,
        grid=(_R,),
        in_specs=[pl.BlockSpec((_B, _D), lambda i: (0, 0))],
        out_specs=pl.BlockSpec(
            block_shape=(1, 1),
            index_map=lambda i: (0, 0),
            memory_space=pltpu.SMEM,
        ),
        out_shape=jax.ShapeDtypeStruct((1, 1), jnp.float32),
        compiler_params=pltpu.CompilerParams(
            dimension_semantics=("arbitrary",)),
    )(y)
    return -(acc[0, 0] / _B)


# speedup vs baseline: 1.1541x; 1.1541x over previous
import jax
import jax.numpy as jnp
from jax.experimental import pallas as pl
from jax.experimental.pallas import tpu as pltpu

_EPS = 1e-8
_B = 4096
_D = 256
_BLK = 512
_R = _B // _BLK
_CH = 512
_NCH = _B // _CH


def _koleo_body(x_ref, acc_ref, y_ref):
    i = pl.program_id(0)

    @pl.when(i == 0)
    def _init():
        x = x_ref[...]
        nrm2 = jax.lax.dot_general(
            x * x, jnp.ones((_D, 1), jnp.float32), (((1,), (0,)), ((), ())),
            preferred_element_type=jnp.float32)
        inv = 1.0 / jnp.maximum(jnp.sqrt(nrm2), _EPS)
        y_ref[...] = (x * inv).astype(jnp.bfloat16)

    xi = y_ref[pl.ds(i * _BLK, _BLK), :]
    diag = (jax.lax.broadcasted_iota(jnp.int32, (_BLK, _CH), 0)
            == jax.lax.broadcasted_iota(jnp.int32, (_BLK, _CH), 1))

    mlanes = jnp.full((_BLK, 128), jnp.float32(-3e38))
    for kk in range(_NCH):
        k = jax.lax.rem(i + kk, _NCH)
        xc = y_ref[pl.ds(k * _CH, _CH), :]
        dch = jax.lax.dot_general(
            xi, xc, (((1,), (1,)), ((), ())),
            preferred_element_type=jnp.float32)
        if kk == 0:
            dch = jnp.where(diag, jnp.float32(-1.0), dch)
        m4 = jnp.maximum(
            jnp.maximum(dch[:, 0:128], dch[:, 128:256]),
            jnp.maximum(dch[:, 256:384], dch[:, 384:512]))
        mlanes = jnp.maximum(mlanes, m4)
    m = jnp.max(mlanes, axis=1, keepdims=True)

    dist2 = 2.0 + _D * _EPS * _EPS - 2.0 * m
    dist = jnp.sqrt(jnp.maximum(dist2, 0.0))
    part = jnp.sum(jnp.log(dist + _EPS))

    @pl.when(i == 0)
    def _first():
        acc_ref[0, 0] = part

    @pl.when(i > 0)
    def _rest():
        acc_ref[0, 0] = acc_ref[0, 0] + part


def kernel(student_output):
    acc = pl.pallas_call(
        _koleo_body,
        grid=(_R,),
        in_specs=[pl.BlockSpec((_B, _D), lambda i: (0, 0))],
        out_specs=pl.BlockSpec(
            block_shape=(1, 1),
            index_map=lambda i: (0, 0),
            memory_space=pltpu.SMEM,
        ),
        out_shape=jax.ShapeDtypeStruct((1, 1), jnp.float32),
        scratch_shapes=[
            pltpu.VMEM((_B, _D), jnp.bfloat16),
        ],
        compiler_params=pltpu.CompilerParams(
            dimension_semantics=("arbitrary",)),
    )(student_output)
    return -(acc[0, 0] / _B)


# speedup vs baseline: 1.2846x; 1.1130x over previous
import jax
import jax.numpy as jnp
from jax.experimental import pallas as pl
from jax.experimental.pallas import tpu as pltpu

_EPS = 1e-8
_B = 4096
_D = 256
_BLK = 1024
_R = _B // _BLK
_CH = _BLK
_NCH = _B // _CH


def _koleo_body(x_ref, acc_ref, y_ref):
    i = pl.program_id(0)

    @pl.when(i == 0)
    def _init():
        x = x_ref[...]
        nrm2 = jax.lax.dot_general(
            x * x, jnp.ones((_D, 1), jnp.float32), (((1,), (0,)), ((), ())),
            preferred_element_type=jnp.float32)
        inv = 1.0 / jnp.maximum(jnp.sqrt(nrm2), _EPS)
        y_ref[...] = (x * inv).astype(jnp.bfloat16)

    xi = y_ref[pl.ds(i * _BLK, _BLK), :]
    diag = (jax.lax.broadcasted_iota(jnp.int32, (_BLK, _CH), 0)
            == jax.lax.broadcasted_iota(jnp.int32, (_BLK, _CH), 1))

    mlanes = jnp.full((_BLK, 128), jnp.float32(-3e38))
    for kk in range(_NCH):
        k = jax.lax.rem(i + kk, _NCH)
        xc = y_ref[pl.ds(k * _CH, _CH), :]
        dch = jax.lax.dot_general(
            xi, xc, (((1,), (1,)), ((), ())),
            preferred_element_type=jnp.float32)
        if kk == 0:
            dch = jnp.where(diag, jnp.float32(-1.0), dch)
        parts = [dch[:, c * 128:(c + 1) * 128] for c in range(_CH // 128)]
        while len(parts) > 1:
            parts = [jnp.maximum(parts[p], parts[p + 1])
                     for p in range(0, len(parts), 2)]
        mlanes = jnp.maximum(mlanes, parts[0])
    m = jnp.max(mlanes, axis=1, keepdims=True)

    dist2 = 2.0 + _D * _EPS * _EPS - 2.0 * m
    dist = jnp.sqrt(jnp.maximum(dist2, 0.0))
    part = jnp.sum(jnp.log(dist + _EPS))

    @pl.when(i == 0)
    def _first():
        acc_ref[0, 0] = part

    @pl.when(i > 0)
    def _rest():
        acc_ref[0, 0] = acc_ref[0, 0] + part


def kernel(student_output):
    acc = pl.pallas_call(
        _koleo_body,
        grid=(_R,),
        in_specs=[pl.BlockSpec((_B, _D), lambda i: (0, 0))],
        out_specs=pl.BlockSpec(
            block_shape=(1, 1),
            index_map=lambda i: (0, 0),
            memory_space=pltpu.SMEM,
        ),
        out_shape=jax.ShapeDtypeStruct((1, 1), jnp.float32),
        scratch_shapes=[
            pltpu.VMEM((_B, _D), jnp.bfloat16),
        ],
        compiler_params=pltpu.CompilerParams(
            dimension_semantics=("arbitrary",)),
    )(student_output)
    return -(acc[0, 0] / _B)


# speedup vs baseline: 1.3506x; 1.0514x over previous
import jax
import jax.numpy as jnp
from jax.experimental import pallas as pl
from jax.experimental.pallas import tpu as pltpu

_EPS = 1e-8
_B = 4096
_D = 256
_BLK = 2048
_R = _B // _BLK
_CH = _BLK
_NCH = _B // _CH


def _koleo_body(x_ref, acc_ref, y_ref):
    i = pl.program_id(0)

    @pl.when(i == 0)
    def _init():
        x = x_ref[...]
        nrm2 = jax.lax.dot_general(
            x * x, jnp.ones((_D, 1), jnp.float32), (((1,), (0,)), ((), ())),
            preferred_element_type=jnp.float32)
        inv = 1.0 / jnp.maximum(jnp.sqrt(nrm2), _EPS)
        y_ref[...] = (x * inv).astype(jnp.bfloat16)

    xi = y_ref[pl.ds(i * _BLK, _BLK), :]
    diag = (jax.lax.broadcasted_iota(jnp.int32, (_BLK, _CH), 0)
            == jax.lax.broadcasted_iota(jnp.int32, (_BLK, _CH), 1))

    mlanes = jnp.full((_BLK, 128), jnp.float32(-3e38))
    for kk in range(_NCH):
        k = jax.lax.rem(i + kk, _NCH)
        xc = y_ref[pl.ds(k * _CH, _CH), :]
        dch = jax.lax.dot_general(
            xi, xc, (((1,), (1,)), ((), ())),
            preferred_element_type=jnp.float32)
        if kk == 0:
            dch = jnp.where(diag, jnp.float32(-1.0), dch)
        parts = [dch[:, c * 128:(c + 1) * 128] for c in range(_CH // 128)]
        while len(parts) > 1:
            parts = [jnp.maximum(parts[p], parts[p + 1])
                     for p in range(0, len(parts), 2)]
        mlanes = jnp.maximum(mlanes, parts[0])
    m = jnp.max(mlanes, axis=1, keepdims=True)

    dist2 = 2.0 + _D * _EPS * _EPS - 2.0 * m
    dist = jnp.sqrt(jnp.maximum(dist2, 0.0))
    part = jnp.sum(jnp.log(dist + _EPS))

    @pl.when(i == 0)
    def _first():
        acc_ref[0, 0] = part

    @pl.when(i > 0)
    def _rest():
        acc_ref[0, 0] = acc_ref[0, 0] + part


def kernel(student_output):
    acc = pl.pallas_call(
        _koleo_body,
        grid=(_R,),
        in_specs=[pl.BlockSpec((_B, _D), lambda i: (0, 0))],
        out_specs=pl.BlockSpec(
            block_shape=(1, 1),
            index_map=lambda i: (0, 0),
            memory_space=pltpu.SMEM,
        ),
        out_shape=jax.ShapeDtypeStruct((1, 1), jnp.float32),
        scratch_shapes=[
            pltpu.VMEM((_B, _D), jnp.bfloat16),
        ],
        compiler_params=pltpu.CompilerParams(
            dimension_semantics=("arbitrary",)),
    )(student_output)
    return -(acc[0, 0] / _B)


# speedup vs baseline: 1.6587x; 1.2281x over previous
import jax
import jax.numpy as jnp
from jax.experimental import pallas as pl
from jax.experimental.pallas import tpu as pltpu

_EPS = 1e-8
_B = 4096
_D = 256
_BLK = 2048
_R = _B // _BLK


def _normalize(x):
    nrm2 = jax.lax.dot_general(
        x * x, jnp.ones((_D, 1), jnp.float32), (((1,), (0,)), ((), ())),
        preferred_element_type=jnp.float32)
    inv = 1.0 / jnp.maximum(jnp.sqrt(nrm2), _EPS)
    return (x * inv).astype(jnp.bfloat16)


def _dot_nt(a, b):
    return jax.lax.dot_general(
        a, b, (((1,), (1,)), ((), ())),
        preferred_element_type=jnp.float32)


def _mask_diag(d):
    diag = (jax.lax.broadcasted_iota(jnp.int32, (_BLK, _BLK), 0)
            == jax.lax.broadcasted_iota(jnp.int32, (_BLK, _BLK), 1))
    return jnp.where(diag, jnp.float32(-1.0), d)


def _rowmax(d):
    parts = [d[:, c * 128:(c + 1) * 128] for c in range(_BLK // 128)]
    while len(parts) > 1:
        parts = [jnp.maximum(parts[p], parts[p + 1])
                 for p in range(0, len(parts), 2)]
    return jnp.max(parts[0], axis=1, keepdims=True)


def _logdist_sum(m):
    dist2 = 2.0 + _D * _EPS * _EPS - 2.0 * m
    dist = jnp.sqrt(jnp.maximum(dist2, 0.0))
    return jnp.sum(jnp.log(dist + _EPS))


def _koleo_body(x_ref, acc_ref, y0_ref, rmax_ref):
    i = pl.program_id(0)

    @pl.when(i == 0)
    def _first_half():
        xn0 = _normalize(x_ref[...])
        y0_ref[...] = xn0
        d00 = _mask_diag(_dot_nt(xn0, xn0))
        rmax_ref[...] = _rowmax(d00)

    @pl.when(i == 1)
    def _second_half():
        xn1 = _normalize(x_ref[...])
        d11 = _mask_diag(_dot_nt(xn1, xn1))
        m1 = _rowmax(d11)
        d10 = _dot_nt(xn1, y0_ref[...])
        m1 = jnp.maximum(m1, _rowmax(d10))
        c0 = jnp.max(d10, axis=0, keepdims=True)
        m0 = jnp.maximum(rmax_ref[...], c0.reshape(_BLK, 1))
        acc_ref[0, 0] = _logdist_sum(m0) + _logdist_sum(m1)


def kernel(student_output):
    acc = pl.pallas_call(
        _koleo_body,
        grid=(_R,),
        in_specs=[pl.BlockSpec((_BLK, _D), lambda i: (i, 0))],
        out_specs=pl.BlockSpec(
            block_shape=(1, 1),
            index_map=lambda i: (0, 0),
            memory_space=pltpu.SMEM,
        ),
        out_shape=jax.ShapeDtypeStruct((1, 1), jnp.float32),
        scratch_shapes=[
            pltpu.VMEM((_BLK, _D), jnp.bfloat16),
            pltpu.VMEM((_BLK, 1), jnp.float32),
        ],
        compiler_params=pltpu.CompilerParams(
            dimension_semantics=("arbitrary",)),
    )(student_output)
    return -(acc[0, 0] / _B)
